# baseline (speedup 1.0000x reference)
import jax
import jax.numpy as jnp
from jax import lax
from jax.experimental import pallas as pl
from jax.experimental.pallas import tpu as pltpu
from jax.experimental.pallas import tpu_sc as plsc

_EMB = 32
_SCALE = float(_EMB) ** 0.5
_LANES = 16
_WINDOW = 1024
_TC_BLOCK_ROWS = 8


def _tc_flatten(xt):
    n_cols, n_rows = xt.shape
    blk = _TC_BLOCK_ROWS * n_rows

    def body(x_ref, o_ref):
        o_ref[...] = x_ref[...].reshape(blk)

    return pl.pallas_call(
        body,
        grid=(n_cols // _TC_BLOCK_ROWS,),
        in_specs=[pl.BlockSpec((_TC_BLOCK_ROWS, n_rows), lambda a: (a, 0))],
        out_specs=pl.BlockSpec((blk,), lambda a: (a,)),
        out_shape=jax.ShapeDtypeStruct((n_cols * n_rows,), jnp.int32),
    )(xt)


def _tc_table_rowmajor(tt):
    emb, vocab = tt.shape
    c_blk = 8192
    grid = -(-vocab // c_blk)
    out_rows = vocab * emb // 128
    r_blk = c_blk * emb // 128
    groups = 128 // emb

    def body(t_ref, o_ref):
        tt_blk = t_ref[...].T.reshape(r_blk, groups, emb) * _SCALE
        for k in range(groups):
            o_ref[:, emb * k:emb * (k + 1)] = tt_blk[:, k, :]

    return pl.pallas_call(
        body,
        grid=(grid,),
        in_specs=[pl.BlockSpec((emb, c_blk), lambda a: (0, a))],
        out_specs=pl.BlockSpec((r_blk, 128), lambda a: (a, 0)),
        out_shape=jax.ShapeDtypeStruct((out_rows, 128), jnp.float32),
    )(tt)


def _gather(idx1d, table, n_cols, n_rows):
    blocks_per_col = n_rows // _WINDOW
    ia_blk = _WINDOW // 128
    mesh = plsc.VectorSubcoreMesh(core_axis_name="c", subcore_axis_name="s")

    @pl.kernel(
        out_type=jax.ShapeDtypeStruct(
            (n_cols, _EMB // 8, n_rows // 128, 8, 128), jnp.float32
        ),
        mesh=mesh,
        scratch_types=[pltpu.VMEM((_WINDOW, _EMB), jnp.float32)],
        compiler_params=pltpu.CompilerParams(
            use_tc_tiling_on_sc=False, needs_layout_passes=False
        ),
    )
    def k(idx_hbm, table_hbm, out_hbm, rows_v):
        def body(idx_vmem, out_vmem):
            pltpu.sync_copy(table_hbm.at[idx_vmem], rows_v)
            packed = out_vmem.at[0]
            iota = lax.iota(jnp.int32, _LANES)

            @plsc.parallel_loop(0, ia_blk * (128 // _LANES), unroll=2)
            def _(q):
                ia = q // (128 // _LANES)
                c = q % (128 // _LANES)
                row_idx = iota + (128 * ia + _LANES * c)
                for ka in range(_EMB // 8):
                    for ks in range(8):
                        col_idx = jnp.full((_LANES,), 8 * ka + ks, jnp.int32)
                        v = plsc.load_gather(rows_v, [row_idx, col_idx])
                        packed.at[ka, ia, ks, pl.ds(_LANES * c, _LANES)][...] = v

        pltpu.emit_pipeline(
            body,
            grid=(n_cols, blocks_per_col),
            in_specs=[
                pl.BlockSpec(
                    (_WINDOW,), lambda j, i: (j * blocks_per_col + i,)
                )
            ],
            out_specs=[
                pl.BlockSpec(
                    (1, _EMB // 8, ia_blk, 8, 128),
                    lambda j, i: (j, 0, i, 0, 0),
                )
            ],
            core_axis_name=("c", "s"),
            dimension_semantics=(pltpu.PARALLEL, pltpu.PARALLEL),
        )(idx_hbm, out_hbm)

    return k(idx1d, table)


def kernel(x, table):
    if x.dtype != jnp.int32:
        x = x.astype(jnp.int32)
    n_rows, n_cols = x.shape
    idx_flat = _tc_flatten(x.T)
    table_lin = _tc_table_rowmajor(table.T).reshape(table.shape)
    out5 = _gather(idx_flat, table_lin, n_cols, n_rows)
    out = jnp.transpose(out5, (2, 4, 0, 1, 3))
    return out.reshape(n_rows, n_cols, _EMB)


# speedup vs baseline: 1.2226x; 1.2226x over previous
import jax
import jax.numpy as jnp
from jax import lax
from jax.experimental import pallas as pl
from jax.experimental.pallas import tpu as pltpu
from jax.experimental.pallas import tpu_sc as plsc

_EMB = 32
_SCALE = float(_EMB) ** 0.5
_LANES = 16
_WINDOW = 1024
_TC_BLOCK_ROWS = 8


def _tc_flatten(xt):
    n_cols, n_rows = xt.shape
    blk = _TC_BLOCK_ROWS * n_rows

    def body(x_ref, o_ref):
        o_ref[...] = x_ref[...].reshape(blk)

    return pl.pallas_call(
        body,
        grid=(n_cols // _TC_BLOCK_ROWS,),
        in_specs=[pl.BlockSpec((_TC_BLOCK_ROWS, n_rows), lambda a: (a, 0))],
        out_specs=pl.BlockSpec((blk,), lambda a: (a,)),
        out_shape=jax.ShapeDtypeStruct((n_cols * n_rows,), jnp.int32),
    )(xt)


def _tc_table_rowmajor(tt):
    emb, vocab = tt.shape
    c_blk = 8192
    grid = -(-vocab // c_blk)
    out_rows = vocab * emb // 128
    r_blk = c_blk * emb // 128
    groups = 128 // emb

    def body(t_ref, o_ref):
        tt_blk = t_ref[...].T.reshape(r_blk, groups, emb) * _SCALE
        o_ref[...] = jnp.concatenate(
            [tt_blk[:, k, :] for k in range(groups)], axis=-1
        )

    return pl.pallas_call(
        body,
        grid=(grid,),
        in_specs=[pl.BlockSpec((emb, c_blk), lambda a: (0, a))],
        out_specs=pl.BlockSpec((r_blk, 128), lambda a: (a, 0)),
        out_shape=jax.ShapeDtypeStruct((out_rows, 128), jnp.float32),
    )(tt)


def _gather(idx1d, table, n_cols, n_rows):
    blocks_per_col = n_rows // _WINDOW
    ia_blk = _WINDOW // 128
    mesh = plsc.VectorSubcoreMesh(core_axis_name="c", subcore_axis_name="s")

    q_blk = _WINDOW * _EMB // 128

    @pl.kernel(
        out_type=jax.ShapeDtypeStruct(
            (n_cols, n_rows * _EMB // 128, 128), jnp.float32
        ),
        mesh=mesh,
        scratch_types=[pltpu.VMEM((_WINDOW, _EMB), jnp.float32)],
        compiler_params=pltpu.CompilerParams(use_tc_tiling_on_sc=False),
    )
    def k(idx_hbm, table_hbm, out_hbm, rows_v):
        def body(idx_vmem, out_vmem):
            pltpu.sync_copy(table_hbm.at[idx_vmem], rows_v)
            packed = out_vmem.at[0]

            @plsc.parallel_loop(0, q_blk, unroll=8)
            def _(q):
                for u in range(128 // _LANES):
                    src = (4 * q + u // 2, pl.ds((u % 2) * _LANES, _LANES))
                    dst = (q, pl.ds(u * _LANES, _LANES))
                    packed.at[dst][...] = rows_v.at[src][...]

        pltpu.emit_pipeline(
            body,
            grid=(n_cols, blocks_per_col),
            in_specs=[
                pl.BlockSpec(
                    (_WINDOW,), lambda j, i: (j * blocks_per_col + i,)
                )
            ],
            out_specs=[pl.BlockSpec((1, q_blk, 128), lambda j, i: (j, i, 0))],
            core_axis_name=("c", "s"),
            dimension_semantics=(pltpu.PARALLEL, pltpu.PARALLEL),
        )(idx_hbm, out_hbm)

    return k(idx1d, table)


def kernel(x, table):
    if x.dtype != jnp.int32:
        x = x.astype(jnp.int32)
    n_rows, n_cols = x.shape
    idx_flat = _tc_flatten(x.T)
    table_lin = _tc_table_rowmajor(table.T).reshape(table.shape)
    out_p = _gather(idx_flat, table_lin, n_cols, n_rows)
    out_t = out_p.reshape(n_cols, n_rows, _EMB)
    return jnp.transpose(out_t, (1, 0, 2))


# speedup vs baseline: 2.0044x; 1.6394x over previous
import jax
import jax.numpy as jnp
from jax.experimental import pallas as pl
from jax.experimental.pallas import tpu as pltpu
from jax.experimental.pallas import tpu_sc as plsc

_EMB = 32
_SCALE = float(_EMB) ** 0.5
_LANES = 16
_WINDOW = 1024
_TC_BLOCK_ROWS = 8
_C_BLK = 8192
_STRIPS = 4
_STRIP = _C_BLK // _STRIPS


def _remap(v):
    blk = v & ~jnp.int32(_C_BLK - 1)
    return blk + ((v & (_STRIP - 1)) << 2) + ((v & (_C_BLK - 1)) >> 11)


def _tc_flatten(xt):
    n_cols, n_rows = xt.shape
    blk = _TC_BLOCK_ROWS * n_rows

    def body(x_ref, o_ref):
        o_ref[...] = _remap(x_ref[...]).reshape(blk)

    return pl.pallas_call(
        body,
        grid=(n_cols // _TC_BLOCK_ROWS,),
        in_specs=[pl.BlockSpec((_TC_BLOCK_ROWS, n_rows), lambda a: (a, 0))],
        out_specs=pl.BlockSpec((blk,), lambda a: (a,)),
        out_shape=jax.ShapeDtypeStruct((n_cols * n_rows,), jnp.int32),
    )(xt)


def _tc_table_perm(tt):
    emb, vocab = tt.shape
    grid = -(-vocab // _C_BLK)
    r_blk = _STRIP
    out_rows = grid * r_blk

    def body(t_ref, o_ref):
        t = t_ref[...]
        stacked = jnp.concatenate(
            [t[:, s * _STRIP:(s + 1) * _STRIP] for s in range(_STRIPS)],
            axis=0,
        )
        o_ref[...] = stacked.T * _SCALE

    return pl.pallas_call(
        body,
        grid=(grid,),
        in_specs=[pl.BlockSpec((emb, _C_BLK), lambda a: (0, a))],
        out_specs=pl.BlockSpec((r_blk, 128), lambda a: (a, 0)),
        out_shape=jax.ShapeDtypeStruct((out_rows, 128), jnp.float32),
    )(tt)


def _gather(idx1d, table, n_cols, n_rows):
    blocks_per_col = n_rows // _WINDOW
    q_blk = _WINDOW * _EMB // 128
    mesh = plsc.VectorSubcoreMesh(core_axis_name="c", subcore_axis_name="s")

    @pl.kernel(
        out_type=jax.ShapeDtypeStruct(
            (n_cols, n_rows * _EMB // 128, 128), jnp.float32
        ),
        mesh=mesh,
        scratch_types=[pltpu.VMEM((_WINDOW, _EMB), jnp.float32)],
        compiler_params=pltpu.CompilerParams(use_tc_tiling_on_sc=False),
    )
    def k(idx_hbm, table_hbm, out_hbm, rows_v):
        def body(idx_vmem, out_vmem):
            pltpu.sync_copy(table_hbm.at[idx_vmem], rows_v)
            packed = out_vmem.at[0]

            @plsc.parallel_loop(0, q_blk, unroll=8)
            def _(q):
                for u in range(128 // _LANES):
                    src = (4 * q + u // 2, pl.ds((u % 2) * _LANES, _LANES))
                    dst = (q, pl.ds(u * _LANES, _LANES))
                    packed.at[dst][...] = rows_v.at[src][...]

        pltpu.emit_pipeline(
            body,
            grid=(n_cols, blocks_per_col),
            in_specs=[
                pl.BlockSpec(
                    (_WINDOW,), lambda j, i: (j * blocks_per_col + i,)
                )
            ],
            out_specs=[pl.BlockSpec((1, q_blk, 128), lambda j, i: (j, i, 0))],
            core_axis_name=("c", "s"),
            dimension_semantics=(pltpu.PARALLEL, pltpu.PARALLEL),
        )(idx_hbm, out_hbm)

    return k(idx1d, table)


def kernel(x, table):
    if x.dtype != jnp.int32:
        x = x.astype(jnp.int32)
    n_rows, n_cols = x.shape
    idx_flat = _tc_flatten(x.T)
    table_perm = _tc_table_perm(table.T)
    table_rows = table_perm.reshape(
        table_perm.shape[0] * 128 // _EMB, _EMB
    )
    out_p = _gather(idx_flat, table_rows, n_cols, n_rows)
    out_t = out_p.reshape(n_cols, n_rows, _EMB)
    return jnp.transpose(out_t, (1, 0, 2))
